# baseline (speedup 1.0000x reference)
import functools

import jax
import jax.numpy as jnp
from jax import lax
from jax.experimental import pallas as pl
from jax.experimental.pallas import tpu as pltpu
from jax.experimental.pallas import tpu_sc as plsc

B, D, N = 8, 256, 40962
P = (N + 6) // 4
K = 7
L = 16
NUM_CORES = 2
NUM_SUBCORES = 16
NW = NUM_CORES * NUM_SUBCORES
ROWS = B * D
ROWS_PER_TILE = ROWS // NW
P_PAD = ((P + L - 1) // L) * L
NBLK = P_PAD // L


def _pool_body(x_hbm, idx_hbm, out_hbm, idx_v, row_v, out_v):
    wid = lax.axis_index("s") * NUM_CORES + lax.axis_index("c")
    base = wid * ROWS_PER_TILE

    pltpu.sync_copy(idx_hbm, idx_v)

    inv_k = jnp.float32(1.0 / K)

    def gather_block(col):
        acc = plsc.load_gather(row_v, [idx_v[pl.ds(col, L)]])
        for j in range(1, K):
            acc = acc + plsc.load_gather(row_v, [idx_v[pl.ds(j * P_PAD + col, L)]])
        return acc * inv_k

    def do_row(r, carry):
        row = base + r
        pltpu.sync_copy(x_hbm.at[row], row_v)

        def blk(i, c):
            col = i * L
            out_v[pl.ds(col, L)] = gather_block(col)
            return c

        lax.fori_loop(0, P // L, blk, 0)
        out_v[pl.ds(P - L, L)] = gather_block(P - L)
        pltpu.sync_copy(out_v, out_hbm.at[row])
        return carry

    lax.fori_loop(0, ROWS_PER_TILE, do_row, 0)


@functools.partial(jax.jit, static_argnames=())
def kernel(x, neigh_orders):
    idx = neigh_orders[:P, :].astype(jnp.int32)
    idx_t = jnp.zeros((K, P_PAD), jnp.int32).at[:, :P].set(idx.T).reshape(-1)
    x2d = x.reshape(ROWS, N)

    pool = pl.kernel(
        _pool_body,
        out_type=jax.ShapeDtypeStruct((ROWS, P), jnp.float32),
        mesh=plsc.VectorSubcoreMesh(
            core_axis_name="c", subcore_axis_name="s",
            num_cores=NUM_CORES, num_subcores=NUM_SUBCORES),
        scratch_types=[
            pltpu.VMEM((K * P_PAD,), jnp.int32),
            pltpu.VMEM((N,), jnp.float32),
            pltpu.VMEM((P,), jnp.float32),
        ],
        compiler_params=pltpu.CompilerParams(needs_layout_passes=False),
    )
    out2d = pool(x2d, idx_t)
    return out2d.reshape(B, D, P)


# speedup vs baseline: 1.0004x; 1.0004x over previous
import functools

import jax
import jax.numpy as jnp
from jax import lax
from jax.experimental import pallas as pl
from jax.experimental.pallas import tpu as pltpu
from jax.experimental.pallas import tpu_sc as plsc

B, D, N = 8, 256, 40962
P = (N + 6) // 4
K = 7
L = 16
NUM_CORES = 2
NUM_SUBCORES = 16
NW = NUM_CORES * NUM_SUBCORES
ROWS = B * D
ROWS_PER_TILE = ROWS // NW
P_PAD = ((P + L - 1) // L) * L
NBLK = P_PAD // L


def _pool_body(x_hbm, idx_hbm, out_hbm, idx_v, row_v, out_v):
    wid = lax.axis_index("s") * NUM_CORES + lax.axis_index("c")
    base = wid * ROWS_PER_TILE

    pltpu.sync_copy(idx_hbm, idx_v)

    inv_k = jnp.float32(1.0 / K)

    def gather_block(col):
        acc = plsc.load_gather(row_v, [idx_v[pl.ds(col, L)]])
        for j in range(1, K):
            acc = acc + plsc.load_gather(row_v, [idx_v[pl.ds(j * P_PAD + col, L)]])
        return acc * inv_k

    def do_row(r, carry):
        row = base + r
        pltpu.sync_copy(x_hbm.at[row], row_v)

        def blk(i, c):
            col = i * L
            out_v[pl.ds(col, L)] = gather_block(col)
            return c

        lax.fori_loop(0, P // L, blk, 0)
        out_v[pl.ds(P - L, L)] = gather_block(P - L)
        pltpu.sync_copy(out_v, out_hbm.at[row])
        return carry

    lax.fori_loop(0, ROWS_PER_TILE, do_row, 0)


@functools.partial(jax.jit, static_argnames=())
def kernel(x, neigh_orders):
    idx = neigh_orders[:P, :].astype(jnp.int32)
    idx_t = jnp.zeros((K, P_PAD), jnp.int32).at[:, :P].set(idx.T).reshape(-1)
    x2d = x.reshape(ROWS, N)

    pool = pl.kernel(
        _pool_body,
        out_type=jax.ShapeDtypeStruct((ROWS, P), jnp.float32),
        mesh=plsc.VectorSubcoreMesh(
            core_axis_name="c", subcore_axis_name="s",
            num_cores=NUM_CORES, num_subcores=NUM_SUBCORES),
        scratch_types=[
            pltpu.VMEM((K * P_PAD,), jnp.int32),
            pltpu.VMEM((N,), jnp.float32),
            pltpu.VMEM((P,), jnp.float32),
        ],
        compiler_params=pltpu.CompilerParams(needs_layout_passes=False, use_tc_tiling_on_sc=True),
    )
    out2d = pool(x2d, idx_t)
    return out2d.reshape(B, D, P)
